# Initial kernel scaffold; baseline (speedup 1.0000x reference)
import jax
import jax.numpy as jnp
from jax.experimental import pallas as pl


def kernel(stu_ids, exer_ids, cpt_ids, labels, adj_correct_se, adj_wrong_se, adj_correct_sc, adj_wrong_sc, emb_stu_cse, emb_exer_c, emb_stu_wse, emb_exer_w, emb_stu_csc, emb_cpt_c, emb_stu_wsc, emb_cpt_w, Wg_se, bg_se, Wg_sc, bg_sc, Wg_stu, bg_stu, Wg_exer, bg_exer, Wg_cpt, bg_cpt, W1, b1, W2, b2, Wk1, bk1, Wk2, bk2):
    raise NotImplementedError("write your pallas kernel here")


# speedup vs baseline: 3.5311x; 3.5311x over previous
import jax
import jax.numpy as jnp
from jax.experimental import pallas as pl

TEMP = 0.1


def _prop(emb_a, emb_b, edge_index, num_layers=2):
    na, nb = emb_a.shape[0], emb_b.shape[0]
    n = na + nb
    x0 = jnp.concatenate([emb_a, emb_b], axis=0)
    src = edge_index[0]
    dst = edge_index[1] + na
    u = jnp.concatenate([src, dst])
    v = jnp.concatenate([dst, src])
    deg = jnp.bincount(u, length=n).astype(jnp.float32)
    rs = jax.lax.rsqrt(jnp.maximum(deg, 1.0))[:, None]
    t1 = jax.ops.segment_sum((rs * x0)[u], v, num_segments=n)
    t2 = jax.ops.segment_sum((rs * rs * t1)[u], v, num_segments=n)
    xf = (x0 + rs * t1 + rs * t2) * (1.0 / 3.0)
    return xf[:na], xf[na:]


def _fuse(e1, e2, Wg, bg):
    h = jnp.concatenate([e1, e2], axis=-1)
    gates = jax.nn.sigmoid(jnp.einsum('nk,gkd->gnd', h, Wg) + bg[:, None, :])
    fused = gates * e1[None] + (1.0 - gates) * e2[None]
    return jnp.mean(fused, axis=0)


def _contrastive(z1, z2, temp):
    z1 = z1 / (jnp.linalg.norm(z1, axis=-1, keepdims=True) + 1e-8)
    z2 = z2 / (jnp.linalg.norm(z2, axis=-1, keepdims=True) + 1e-8)
    sim = (z1 @ z2.T) / temp
    logp = jax.nn.log_softmax(sim, axis=-1)
    idx = jnp.arange(z1.shape[0])
    return -jnp.mean(logp[idx, idx])


def kernel(stu_ids, exer_ids, cpt_ids, labels, adj_correct_se, adj_wrong_se,
           adj_correct_sc, adj_wrong_sc,
           emb_stu_cse, emb_exer_c, emb_stu_wse, emb_exer_w,
           emb_stu_csc, emb_cpt_c, emb_stu_wsc, emb_cpt_w,
           Wg_se, bg_se, Wg_sc, bg_sc, Wg_stu, bg_stu, Wg_exer, bg_exer,
           Wg_cpt, bg_cpt, W1, b1, W2, b2, Wk1, bk1, Wk2, bk2):
    stu_c_se, exer_c = _prop(emb_stu_cse, emb_exer_c, adj_correct_se)
    stu_w_se, exer_w = _prop(emb_stu_wse, emb_exer_w, adj_wrong_se)
    stu_c_sc, cpt_c = _prop(emb_stu_csc, emb_cpt_c, adj_correct_sc)
    stu_w_sc, cpt_w = _prop(emb_stu_wsc, emb_cpt_w, adj_wrong_sc)

    exer_c_b = exer_c[exer_ids]
    exer_w_b = exer_w[exer_ids]
    stu_se_b = _fuse(stu_c_se[stu_ids], stu_w_se[stu_ids], Wg_se, bg_se)
    stu_sc_b = _fuse(stu_c_sc[stu_ids], stu_w_sc[stu_ids], Wg_sc, bg_sc)
    b_stu = _fuse(stu_se_b, stu_sc_b, Wg_stu, bg_stu)
    b_exer = _fuse(exer_c_b, exer_w_b, Wg_exer, bg_exer)
    cpt_final = _fuse(cpt_c, cpt_w, Wg_cpt, bg_cpt)

    c_exer = _contrastive(exer_c_b, exer_w_b, TEMP)
    cpt_batch = cpt_ids[:, 0]
    c_cpt = _contrastive(cpt_c[cpt_batch], cpt_w[cpt_batch], TEMP)

    b_cpt = cpt_final[cpt_ids]
    cpt_mean = jnp.mean(b_cpt, axis=1)
    h = jnp.concatenate([b_stu, b_exer, cpt_mean], axis=-1)
    h = jax.nn.relu(h @ W1 + b1)
    predictions = jax.nn.sigmoid(h @ W2 + b2)[:, 0]
    kh = jax.nn.relu(b_stu @ Wk1 + bk1)
    knowledge_state = jax.nn.sigmoid(kh @ Wk2 + bk2)
    return (predictions, knowledge_state, c_exer, c_cpt)


# speedup vs baseline: 18.8560x; 5.3399x over previous
import functools

import jax
import jax.numpy as jnp
from jax import lax
from jax.experimental import pallas as pl
from jax.experimental.pallas import tpu as pltpu
from jax.experimental.pallas import tpu_sc as plsc

TEMP = 0.1
NC = 2
NS = 16
LN = 16

BLK = 2048
RPB = BLK // 128


@functools.lru_cache(maxsize=None)
def _make_deg_kernel(n_acc, e_pad):
    e_pt = e_pad // (NC * NS)
    nblk = e_pt // 1024
    assert nblk * 1024 == e_pt
    slc = n_acc // NS
    mesh = plsc.VectorSubcoreMesh(core_axis_name="c", subcore_axis_name="s")

    @functools.partial(
        pl.kernel, mesh=mesh,
        compiler_params=pltpu.CompilerParams(use_tc_tiling_on_sc=False, needs_layout_passes=False),
        out_type=jax.ShapeDtypeStruct((NC * NS * n_acc,), jnp.float32),
        scratch_types=[
            pltpu.VMEM((1024,), jnp.int32),
            pltpu.VMEM((n_acc,), jnp.float32),
        ],
    )
    def k(u_hbm, out_hbm, u_vm, hist):
        c = lax.axis_index("c")
        s = lax.axis_index("s")
        w = c * NS + s

        def zh(i, _):
            hist[pl.ds(i * LN, LN)] = jnp.zeros((LN,), jnp.float32)
            return 0
        lax.fori_loop(0, n_acc // LN, zh, 0)

        ones = jnp.ones((LN,), jnp.float32)

        def blk(b, _):
            pltpu.sync_copy(u_hbm.at[pl.ds(w * e_pt + b * 1024, 1024)], u_vm)
            for j in range(1024 // LN):
                iv = u_vm[pl.ds(j * LN, LN)]
                plsc.addupdate_scatter(hist, [iv], ones)
            return 0
        lax.fori_loop(0, nblk, blk, 0)

        pltpu.sync_copy(hist, out_hbm.at[pl.ds(w * n_acc, n_acc)])

    return k


@functools.lru_cache(maxsize=None)
def _make_scatter_kernel(n_acc, e_pad):
    e_pc = e_pad // NS
    nblk = e_pc // BLK
    assert nblk * BLK == e_pc
    rows_pc = e_pc // 128
    slc = n_acc // NS
    mesh = plsc.VectorSubcoreMesh(core_axis_name="c", subcore_axis_name="s")

    @functools.partial(
        pl.kernel, mesh=mesh,
        compiler_params=pltpu.CompilerParams(use_tc_tiling_on_sc=False, needs_layout_passes=False),
        out_type=jax.ShapeDtypeStruct((4 * n_acc, LN), jnp.float32),
        scratch_types=[
            pltpu.VMEM((RPB, 128), jnp.int32),
            pltpu.VMEM((RPB, 128), jnp.int32),
            pltpu.VMEM((RPB, 128), jnp.int32),
            pltpu.VMEM((BLK, LN), jnp.float32),
            pltpu.VMEM_SHARED((n_acc, LN), jnp.float32),
            pltpu.SemaphoreType.DMA,
            pltpu.SemaphoreType.DMA,
        ],
    )
    def k(y_hbm, u_hbm, v_hbm, out_hbm, u_vm, uo_vm, v_vm, rows_vm, acc, sem_g, sem_s):
        c = lax.axis_index("c")
        s = lax.axis_index("s")

        nz_full = slc // BLK
        for qi in range(2):
            q = 2 * c + qi
            qoff = q * n_acc

            def zrows(i, _):
                rows_vm[i, :] = jnp.zeros((LN,), jnp.float32)
                return 0
            lax.fori_loop(0, BLK, zrows, 0)
            for zi in range(nz_full):
                pltpu.sync_copy(rows_vm, acc.at[pl.ds(s * slc + zi * BLK, BLK)])
            rem = slc - nz_full * BLK
            if rem:
                pltpu.sync_copy(rows_vm.at[pl.ds(0, rem)],
                                acc.at[pl.ds(s * slc + nz_full * BLK, rem)])
            plsc.subcore_barrier()

            def blk(b, _):
                row0 = s * rows_pc + b * RPB
                pltpu.sync_copy(u_hbm.at[pl.ds(row0, RPB)], u_vm)
                pltpu.sync_copy(v_hbm.at[pl.ds(row0, RPB)], v_vm)
                for j in range(RPB):
                    for l in range(128 // LN):
                        uo_vm[j, pl.ds(l * LN, LN)] = (
                            u_vm[j, pl.ds(l * LN, LN)] + qoff)
                gathers = [
                    pltpu.async_copy(
                        y_hbm.at[uo_vm.at[j]],
                        rows_vm.at[pl.ds(j * 128, 128)], sem_g)
                    for j in range(RPB)
                ]
                for g in gathers:
                    g.wait()
                scatters = [
                    pltpu.async_copy(
                        rows_vm.at[pl.ds(j * 128, 128)],
                        acc.at[v_vm.at[j]], sem_s, add=True)
                    for j in range(RPB)
                ]
                for sc in scatters:
                    sc.wait()
                return 0
            lax.fori_loop(0, nblk, blk, 0)

            plsc.subcore_barrier()
            pltpu.sync_copy(acc.at[pl.ds(s * slc, slc)],
                            out_hbm.at[pl.ds(qoff + s * slc, slc)])
            plsc.subcore_barrier()

    return k


def _quarter(x_pad):
    n_acc = x_pad.shape[0]
    return jnp.transpose(x_pad.reshape(n_acc, 4, LN), (1, 0, 2)).reshape(4 * n_acc, LN)


def _unquarter_rows(tab, idx):
    n_acc = tab.shape[0] // 4
    q = tab.reshape(4, n_acc, LN)[:, idx, :]
    return jnp.transpose(q, (1, 0, 2)).reshape(idx.shape[0], 64)


def _propagate(emb_a, emb_b, edge_index, n_acc):
    na, nb = emb_a.shape[0], emb_b.shape[0]
    n = na + nb
    e = edge_index.shape[1]
    src = edge_index[0]
    dst = edge_index[1] + na
    u = jnp.concatenate([src, dst])
    v = jnp.concatenate([dst, src])
    e_pad = ((2 * e + NS * BLK - 1) // (NS * BLK)) * (NS * BLK)
    pad = e_pad - 2 * e
    u = jnp.concatenate([u, jnp.full((pad,), n, jnp.int32)])
    v = jnp.concatenate([v, jnp.full((pad,), n, jnp.int32)])
    u2d = u.reshape(e_pad // 128, 128)
    v2d = v.reshape(e_pad // 128, 128)

    deg_part = _make_deg_kernel(n_acc, e_pad)(u)
    deg = deg_part.reshape(NC * NS, n_acc).sum(axis=0)
    rs = lax.rsqrt(jnp.maximum(deg, 1.0))[:, None]

    x0 = jnp.zeros((n_acc, 64), jnp.float32).at[:n].set(
        jnp.concatenate([emb_a, emb_b], axis=0))
    rs4 = jnp.tile(rs, (4, 1))

    scat = _make_scatter_kernel(n_acc, e_pad)
    y0 = _quarter(x0 * rs)
    t1 = scat(y0, u2d, v2d)
    t2 = scat(t1 * (rs4 * rs4), u2d, v2d)
    return x0, t1, t2, rs


def _final_rows(x0, t1, t2, rs, idx):
    r = rs[idx]
    return (x0[idx] + r * (_unquarter_rows(t1, idx) + _unquarter_rows(t2, idx))) * (1.0 / 3.0)


def _fuse(e1, e2, Wg, bg):
    h = jnp.concatenate([e1, e2], axis=-1)
    gates = jax.nn.sigmoid(jnp.einsum('nk,gkd->gnd', h, Wg) + bg[:, None, :])
    fused = gates * e1[None] + (1.0 - gates) * e2[None]
    return jnp.mean(fused, axis=0)


def _contrastive(z1, z2, temp):
    z1 = z1 / (jnp.linalg.norm(z1, axis=-1, keepdims=True) + 1e-8)
    z2 = z2 / (jnp.linalg.norm(z2, axis=-1, keepdims=True) + 1e-8)
    sim = (z1 @ z2.T) / temp
    logp = jax.nn.log_softmax(sim, axis=-1)
    idx = jnp.arange(z1.shape[0])
    return -jnp.mean(logp[idx, idx])


def kernel(stu_ids, exer_ids, cpt_ids, labels, adj_correct_se, adj_wrong_se,
           adj_correct_sc, adj_wrong_sc,
           emb_stu_cse, emb_exer_c, emb_stu_wse, emb_exer_w,
           emb_stu_csc, emb_cpt_c, emb_stu_wsc, emb_cpt_w,
           Wg_se, bg_se, Wg_sc, bg_sc, Wg_stu, bg_stu, Wg_exer, bg_exer,
           Wg_cpt, bg_cpt, W1, b1, W2, b2, Wk1, bk1, Wk2, bk2):
    S = emb_stu_cse.shape[0]
    C = emb_cpt_c.shape[0]
    N_SE = 70144
    N_SC = 51200

    p_cse = _propagate(emb_stu_cse, emb_exer_c, adj_correct_se, N_SE)
    p_wse = _propagate(emb_stu_wse, emb_exer_w, adj_wrong_se, N_SE)
    p_csc = _propagate(emb_stu_csc, emb_cpt_c, adj_correct_sc, N_SC)
    p_wsc = _propagate(emb_stu_wsc, emb_cpt_w, adj_wrong_sc, N_SC)

    exer_nodes = S + exer_ids
    cpt_nodes = S + jnp.arange(C, dtype=jnp.int32)

    stu_c_se_b = _final_rows(*p_cse, stu_ids)
    stu_w_se_b = _final_rows(*p_wse, stu_ids)
    stu_c_sc_b = _final_rows(*p_csc, stu_ids)
    stu_w_sc_b = _final_rows(*p_wsc, stu_ids)
    exer_c_b = _final_rows(*p_cse, exer_nodes)
    exer_w_b = _final_rows(*p_wse, exer_nodes)
    cpt_c_t = _final_rows(*p_csc, cpt_nodes)
    cpt_w_t = _final_rows(*p_wsc, cpt_nodes)

    stu_se_b = _fuse(stu_c_se_b, stu_w_se_b, Wg_se, bg_se)
    stu_sc_b = _fuse(stu_c_sc_b, stu_w_sc_b, Wg_sc, bg_sc)
    b_stu = _fuse(stu_se_b, stu_sc_b, Wg_stu, bg_stu)
    b_exer = _fuse(exer_c_b, exer_w_b, Wg_exer, bg_exer)
    cpt_final = _fuse(cpt_c_t, cpt_w_t, Wg_cpt, bg_cpt)

    c_exer = _contrastive(exer_c_b, exer_w_b, TEMP)
    cpt_batch = cpt_ids[:, 0]
    c_cpt = _contrastive(cpt_c_t[cpt_batch], cpt_w_t[cpt_batch], TEMP)

    b_cpt = cpt_final[cpt_ids]
    cpt_mean = jnp.mean(b_cpt, axis=1)
    h = jnp.concatenate([b_stu, b_exer, cpt_mean], axis=-1)
    h = jax.nn.relu(h @ W1 + b1)
    predictions = jax.nn.sigmoid(h @ W2 + b2)[:, 0]
    kh = jax.nn.relu(b_stu @ Wk1 + bk1)
    knowledge_state = jax.nn.sigmoid(kh @ Wk2 + bk2)
    return (predictions, knowledge_state, c_exer, c_cpt)
